# baseline (speedup 1.0000x reference)
import jax
import jax.numpy as jnp
from jax.experimental import pallas as pl
from jax.experimental.pallas import tpu as pltpu

BM = 512
NK = 2


def _graph_layer_kernel(h_ref, k_ref, b_ref, adj_ref, out_ref, hk_ref):
    j = pl.program_id(1)
    bk = h_ref.shape[0]

    @pl.when(pl.program_id(0) == 0)
    def _compute_projection_half():
        hk_ref[pl.ds(j * bk, bk), :] = jnp.dot(
            h_ref[...], k_ref[...], preferred_element_type=jnp.float32)

    partial = jnp.dot(adj_ref[...], hk_ref[pl.ds(j * bk, bk), :],
                      preferred_element_type=jnp.float32)

    @pl.when(j == 0)
    def _init():
        out_ref[...] = partial + b_ref[...]

    @pl.when(j != 0)
    def _accum():
        out_ref[...] += partial


def kernel(H, adj, kernels, biases, v_rows, v_cols):
    del v_rows, v_cols
    num_heads, d_in, size_per_head = kernels.shape
    hidden = num_heads * size_per_head
    k_full = jnp.transpose(kernels, (1, 0, 2)).reshape(d_in, hidden)
    b_full = biases.reshape(1, hidden)

    n = adj.shape[0]
    bk = n // NK
    grid = (n // BM, NK)
    out = pl.pallas_call(
        _graph_layer_kernel,
        grid=grid,
        in_specs=[
            pl.BlockSpec((bk, d_in), lambda i, j: (jnp.minimum(i + j, NK - 1), 0)),
            pl.BlockSpec((d_in, hidden), lambda i, j: (0, 0)),
            pl.BlockSpec((1, hidden), lambda i, j: (0, 0)),
            pl.BlockSpec((BM, bk), lambda i, j: (i, j)),
        ],
        out_specs=pl.BlockSpec((BM, hidden), lambda i, j: (i, 0)),
        out_shape=jax.ShapeDtypeStruct((n, hidden), jnp.float32),
        scratch_shapes=[pltpu.VMEM((n, hidden), jnp.float32)],
        compiler_params=pltpu.CompilerParams(
            dimension_semantics=("arbitrary", "arbitrary"),
        ),
    )(H, k_full, b_full, adj)
    return out
